# baseline (speedup 1.0000x reference)
import functools

import jax
import jax.numpy as jnp
from jax.experimental import pallas as pl
from jax.experimental.pallas import tpu as pltpu


def _decode(key):
    s = jnp.where(key < 0, key ^ jnp.int32(0x7FFFFFFF), key)
    return jax.lax.bitcast_convert_type(s, jnp.float32)


def _gate_kernel(x_ref, w_ref, idx_ref, wgt_ref):
    logits = jnp.dot(x_ref[:, :], w_ref[:, :],
                     preferred_element_type=jnp.float32)
    e = logits.shape[-1]
    s = jax.lax.bitcast_convert_type(logits, jnp.int32)
    key = jnp.where(s < 0, s ^ jnp.int32(0x7FFFFFFF), s)
    inv_iota = jax.lax.broadcasted_iota(jnp.int32, logits.shape, 1)
    keyi = (key & jnp.int32(-e)) | (jnp.int32(e - 1) - inv_iota)
    k1 = jnp.max(keyi, axis=1, keepdims=True)
    masked = jnp.where(keyi == k1, jnp.int32(-0x80000000), keyi)
    k2 = jnp.max(masked, axis=1, keepdims=True)
    i1 = jnp.int32(e - 1) - (k1 & jnp.int32(e - 1))
    i2 = jnp.int32(e - 1) - (k2 & jnp.int32(e - 1))
    m1 = _decode(k1 & jnp.int32(-e))
    m2 = _decode(k2 & jnp.int32(-e))
    t = jnp.exp(m2 - m1)
    w1 = 1.0 / (1.0 + t)
    idx_ref[:, :] = jnp.concatenate([i1, i2], axis=1)
    wgt_ref[:, :] = jnp.concatenate([w1, 1.0 - w1], axis=1)


@functools.partial(jax.jit, static_argnames=("tile",))
def _gate(xf, wt, tile):
    n, d = xf.shape
    e = wt.shape[1]
    idx, wgt = pl.pallas_call(
        _gate_kernel,
        grid=(n // tile,),
        in_specs=[
            pl.BlockSpec((tile, d), lambda i: (i, 0)),
            pl.BlockSpec((d, e), lambda i: (0, 0)),
        ],
        out_specs=[
            pl.BlockSpec((tile, 2), lambda i: (i, 0)),
            pl.BlockSpec((tile, 2), lambda i: (i, 0)),
        ],
        out_shape=[
            jax.ShapeDtypeStruct((n, 2), jnp.int32),
            jax.ShapeDtypeStruct((n, 2), jnp.float32),
        ],
        compiler_params=pltpu.CompilerParams(
            dimension_semantics=("arbitrary",),
        ),
    )(xf, wt)
    return idx, wgt


def kernel(x, w_gate):
    b, t, d = x.shape
    xf = x.reshape(b * t, d)
    wt = w_gate.T
    idx, wgt = _gate(xf, wt, tile=4096)
    return idx.reshape(b, t, 2), wgt.reshape(b, t, 2)


# speedup vs baseline: 1.0827x; 1.0827x over previous
import functools

import jax
import jax.numpy as jnp
from jax.experimental import pallas as pl
from jax.experimental.pallas import tpu as pltpu


def _gate_kernel(x_ref, w_ref, idx_ref, wgt_ref):
    logits = jnp.dot(x_ref[:, :], w_ref[:, :],
                     preferred_element_type=jnp.float32)
    e = logits.shape[-1]
    s = jax.lax.bitcast_convert_type(logits, jnp.int32)
    inv = jnp.int32(e - 1) - jax.lax.broadcasted_iota(jnp.int32, s.shape, 1)
    keyf = jax.lax.bitcast_convert_type((s & jnp.int32(-e)) | inv,
                                        jnp.float32)
    k1 = jnp.max(keyf, axis=1, keepdims=True)
    masked = jnp.where(keyf == k1, -jnp.inf, keyf)
    k2 = jnp.max(masked, axis=1, keepdims=True)
    b1 = jax.lax.bitcast_convert_type(k1, jnp.int32)
    b2 = jax.lax.bitcast_convert_type(k2, jnp.int32)
    i1 = jnp.int32(e - 1) - (b1 & jnp.int32(e - 1))
    i2 = jnp.int32(e - 1) - (b2 & jnp.int32(e - 1))
    m1 = jax.lax.bitcast_convert_type(b1 & jnp.int32(-e), jnp.float32)
    m2 = jax.lax.bitcast_convert_type(b2 & jnp.int32(-e), jnp.float32)
    t = jnp.exp(m2 - m1)
    w1 = 1.0 / (1.0 + t)
    idx_ref[:, :] = jnp.concatenate([i1, i2], axis=1)
    wgt_ref[:, :] = jnp.concatenate([w1, 1.0 - w1], axis=1)


@functools.partial(jax.jit, static_argnames=("tile",))
def _gate(xf, wt, tile):
    n, d = xf.shape
    e = wt.shape[1]
    idx, wgt = pl.pallas_call(
        _gate_kernel,
        grid=(n // tile,),
        in_specs=[
            pl.BlockSpec((tile, d), lambda i: (i, 0)),
            pl.BlockSpec((d, e), lambda i: (0, 0)),
        ],
        out_specs=[
            pl.BlockSpec((tile, 2), lambda i: (i, 0)),
            pl.BlockSpec((tile, 2), lambda i: (i, 0)),
        ],
        out_shape=[
            jax.ShapeDtypeStruct((n, 2), jnp.int32),
            jax.ShapeDtypeStruct((n, 2), jnp.float32),
        ],
        compiler_params=pltpu.CompilerParams(
            dimension_semantics=("arbitrary",),
        ),
    )(xf, wt)
    return idx, wgt


def kernel(x, w_gate):
    b, t, d = x.shape
    xf = x.reshape(b * t, d)
    wt = w_gate.T
    idx, wgt = _gate(xf, wt, tile=4096)
    return idx.reshape(b, t, 2), wgt.reshape(b, t, 2)
